# baseline (speedup 1.0000x reference)
import math

import jax
import jax.numpy as jnp
from jax import lax
from jax.experimental import pallas as pl
from jax.experimental.pallas import tpu as pltpu
from jax.experimental.pallas import tpu_sc as plsc

_NC = 2
_NS = 16
_L = 16
_CH = 256
_SUP = 2048
_R = 4
_NB = 2


def _chunks(total):
    out, off = [], 0
    while off < total:
        sz = min(_CH, total - off)
        out.append((off, sz))
        off += sz
    return out


def _make_seg_sum(n, e_pad, d, with_count):
    nh = n // _NC
    nb = nh // _NB
    acc_rows = nb + _L
    per_tile = e_pad // _NS
    nsup = per_tile // _SUP
    ngrp = _SUP // _CH // _R
    cs = -(-(acc_rows // _NS) // 8) * 8
    cs_last = acc_rows - (_NS - 1) * cs
    cw_last = nb - (_NS - 1) * cs

    mesh = plsc.VectorSubcoreMesh(core_axis_name="c", subcore_axis_name="s")
    out_type = [jax.ShapeDtypeStruct((n, d), jnp.float32)]
    if with_count:
        out_type.append(jax.ShapeDtypeStruct((n,), jnp.float32))

    scratch = (
        [pltpu.VMEM((_SUP,), jnp.int32)] * 2
        + [pltpu.VMEM((_CH,), jnp.int32) for _ in range(_R)]
        + [pltpu.VMEM((_CH, d), jnp.float32) for _ in range(_R)]
        + [pltpu.SemaphoreType.DMA] * _R
        + [pltpu.SemaphoreType.DMA] * _R
        + [pltpu.VMEM_SHARED((acc_rows, d), jnp.float32)]
    )
    if with_count:
        scratch = scratch + (
            [pltpu.VMEM((_CH,), jnp.float32)]
            + [pltpu.SemaphoreType.DMA] * _R
            + [pltpu.VMEM((cs,), jnp.float32)]
            + [pltpu.VMEM_SHARED((acc_rows,), jnp.float32)]
        )

    def body(*refs):
        it = iter(refs)
        h_hbm = next(it)
        src_hbm = next(it)
        dst_hbm = next(it)
        z2d = next(it)
        z1d = next(it) if with_count else None
        sums_hbm = next(it)
        cnt_hbm = next(it) if with_count else None
        src_v = next(it)
        dst_v = next(it)
        locs = [next(it) for _ in range(_R)]
        rowss = [next(it) for _ in range(_R)]
        gsems = [next(it) for _ in range(_R)]
        ssems = [next(it) for _ in range(_R)]
        acc = next(it)
        if with_count:
            ones_v = next(it)
            csems = [next(it) for _ in range(_R)]
            wb1 = next(it)
            cntacc = next(it)

        c = lax.axis_index("c")
        s = lax.axis_index("s")

        if with_count:
            for i in range(_CH // _L):
                ones_v[pl.ds(i * _L, _L)] = jnp.ones((_L,), jnp.float32)

        def pass_body(p, carry):
            n0 = c * nh + p * nb

            pltpu.sync_copy(z2d, rowss[0])
            if with_count:
                pltpu.sync_copy(z1d, wb1)

            @pl.when(s < _NS - 1)
            def _():
                for off, sz in _chunks(cs):
                    pltpu.sync_copy(rowss[0].at[pl.ds(0, sz)],
                                    acc.at[pl.ds(s * cs + off, sz)])
                if with_count:
                    pltpu.sync_copy(wb1, cntacc.at[pl.ds(s * cs, cs)])

            @pl.when(s == _NS - 1)
            def _():
                for off, sz in _chunks(cs_last):
                    pltpu.sync_copy(rowss[0].at[pl.ds(0, sz)],
                                    acc.at[pl.ds((_NS - 1) * cs + off, sz)])
                if with_count:
                    pltpu.sync_copy(wb1.at[pl.ds(0, cs_last)],
                                    cntacc.at[pl.ds((_NS - 1) * cs, cs_last)])

            plsc.subcore_barrier()

            def sup_body(u, carry2):
                off = s * per_tile + u * _SUP
                pltpu.sync_copy(src_hbm.at[pl.ds(off, _SUP)], src_v)
                pltpu.sync_copy(dst_hbm.at[pl.ds(off, _SUP)], dst_v)
                for g in range(ngrp):
                    gds = []
                    for b in range(_R):
                        j = g * _R + b
                        def _drain():
                            pltpu.make_async_copy(
                                rowss[b], acc.at[locs[b]], ssems[b]).wait()
                            if with_count:
                                pltpu.make_async_copy(
                                    ones_v, cntacc.at[locs[b]], csems[b]).wait()
                        if g == 0:
                            pl.when(u > 0)(_drain)
                        else:
                            _drain()
                        for i in range(_CH // _L):
                            o = j * _CH + i * _L
                            dvec = dst_v[pl.ds(o, _L)]
                            m = (dvec >= n0) & (dvec < n0 + nb)
                            locs[b][pl.ds(i * _L, _L)] = jnp.where(m, dvec - n0, nb)
                        gds.append(pltpu.async_copy(
                            h_hbm.at[src_v.at[pl.ds(j * _CH, _CH)]],
                            rowss[b], gsems[b]))
                    for b in range(_R):
                        gds[b].wait()
                        pltpu.async_copy(rowss[b], acc.at[locs[b]], ssems[b],
                                         add=True)
                        if with_count:
                            pltpu.async_copy(ones_v, cntacc.at[locs[b]],
                                             csems[b], add=True)
                return carry2

            lax.fori_loop(0, nsup, sup_body, 0)

            for b in range(_R):
                pltpu.make_async_copy(rowss[b], acc.at[locs[b]], ssems[b]).wait()
                if with_count:
                    pltpu.make_async_copy(ones_v, cntacc.at[locs[b]],
                                          csems[b]).wait()
            plsc.subcore_barrier()

            @pl.when(s < _NS - 1)
            def _():
                for off, sz in _chunks(cs):
                    pltpu.sync_copy(acc.at[pl.ds(s * cs + off, sz)],
                                    rowss[0].at[pl.ds(0, sz)])
                    pltpu.sync_copy(rowss[0].at[pl.ds(0, sz)],
                                    sums_hbm.at[pl.ds(n0 + s * cs + off, sz)])
                if with_count:
                    pltpu.sync_copy(cntacc.at[pl.ds(s * cs, cs)], wb1)
                    pltpu.sync_copy(wb1, cnt_hbm.at[pl.ds(n0 + s * cs, cs)])

            @pl.when(s == _NS - 1)
            def _():
                for off, sz in _chunks(cw_last):
                    pltpu.sync_copy(acc.at[pl.ds((_NS - 1) * cs + off, sz)],
                                    rowss[0].at[pl.ds(0, sz)])
                    pltpu.sync_copy(
                        rowss[0].at[pl.ds(0, sz)],
                        sums_hbm.at[pl.ds(n0 + (_NS - 1) * cs + off, sz)])
                if with_count:
                    pltpu.sync_copy(cntacc.at[pl.ds((_NS - 1) * cs, cw_last)],
                                    wb1.at[pl.ds(0, cw_last)])
                    pltpu.sync_copy(
                        wb1.at[pl.ds(0, cw_last)],
                        cnt_hbm.at[pl.ds(n0 + (_NS - 1) * cs, cw_last)])
            return carry

        lax.fori_loop(0, _NB, pass_body, 0)

    return pl.kernel(
        body, out_type=out_type, mesh=mesh, scratch_types=scratch,
        compiler_params=pltpu.CompilerParams(use_tc_tiling_on_sc=False),
    )


def _minmax_body(ev_ref, min_ref, max_ref):
    i = pl.program_id(0)
    bmin = jnp.min(ev_ref[...], axis=0, keepdims=True)
    bmax = jnp.max(ev_ref[...], axis=0, keepdims=True)

    @pl.when(i == 0)
    def _():
        min_ref[...] = bmin
        max_ref[...] = bmax

    @pl.when(i > 0)
    def _():
        min_ref[...] = jnp.minimum(min_ref[...], bmin)
        max_ref[...] = jnp.maximum(max_ref[...], bmax)


def _lam_body(evals_ref, alpha_ref, emin_ref, emax_ref,
              lam_ref, lams_ref, pb_ref):
    k = alpha_ref.shape[0]
    ev = evals_ref[...]
    lmin = jnp.min(ev)
    lmax = jnp.max(ev)
    x = (ev - lmin) / (lmax - lmin) * 2.0 - 1.0
    acc = jnp.ones_like(x) * alpha_ref[0:1, :] + x * alpha_ref[1:2, :]

    def step(kk, carry):
        tkm1, tk, a = carry
        tk1 = 2.0 * x * tk - tkm1
        a = a + tk1 * alpha_ref[pl.ds(kk, 1), :]
        return (tk, tk1, a)

    _, _, acc = lax.fori_loop(2, k, step, (jnp.ones_like(x), x, acc))
    lam_ref[...] = acc
    emin = emin_ref[...]
    emax = emax_ref[...]
    a_col = 2.0 / (emax - emin)
    b_col = -2.0 * emin / (emax - emin) - 1.0
    lams_ref[...] = acc * a_col
    pb_ref[...] = jnp.sum(acc * b_col, axis=0, keepdims=True)


def _h0_body(x_ref, ev_ref, wf_ref, bf_ref, lams_ref, pb_ref, h0_ref):
    feat = lax.dot_general(x_ref[...], wf_ref[...], (((1,), (1,)), ((), ())),
                           preferred_element_type=jnp.float32) + bf_ref[...]
    pos = lax.dot_general(ev_ref[...], lams_ref[...], (((1,), (0,)), ((), ())),
                          preferred_element_type=jnp.float32) + pb_ref[...]
    h0_ref[...] = jnp.concatenate([feat, pos], axis=1)


def _sage_body(sums_ref, cnt_ref, h_ref, wl_ref, bl_ref, wr_ref, out_ref):
    agg = sums_ref[...] * (1.0 / jnp.maximum(cnt_ref[...], 1.0))
    t = (lax.dot_general(agg, wl_ref[...], (((1,), (1,)), ((), ())),
                         preferred_element_type=jnp.float32) + bl_ref[...]
         + lax.dot_general(h_ref[...], wr_ref[...], (((1,), (1,)), ((), ())),
                           preferred_element_type=jnp.float32))
    out_ref[...] = jnp.maximum(t, 0.0)


def _final_body(sums_ref, cnt_ref, h_ref, wl_ref, bl_ref, wr_ref,
                wo_ref, bo_ref, out_ref):
    agg = sums_ref[...] * (1.0 / jnp.maximum(cnt_ref[...], 1.0))
    t = (lax.dot_general(agg, wl_ref[...], (((1,), (1,)), ((), ())),
                         preferred_element_type=jnp.float32) + bl_ref[...]
         + lax.dot_general(h_ref[...], wr_ref[...], (((1,), (1,)), ((), ())),
                           preferred_element_type=jnp.float32))
    h2 = jnp.maximum(t, 0.0)
    z = lax.dot_general(h2, wo_ref[...], (((1,), (1,)), ((), ())),
                        preferred_element_type=jnp.float32) + bo_ref[...]
    zs = z - jnp.max(z, axis=1, keepdims=True)
    out_ref[...] = zs - jnp.log(jnp.sum(jnp.exp(zs), axis=1, keepdims=True))


def _full(shape):
    return pl.BlockSpec(shape, lambda i: (0,) * len(shape))


def kernel(x, eigenvectors, eigenvalues, edge_index, W_feat, b_feat, alpha,
           Wl1, bl1, Wr1, Wl2, bl2, Wr2, W_out, b_out):
    n, in_dim = x.shape
    neig = eigenvectors.shape[1]
    hd = W_feat.shape[0]
    pd = alpha.shape[1]
    d = hd + pd
    out_dim = W_out.shape[0]
    e = edge_index.shape[1]

    bn = 2000
    grid = (n // bn,)

    emin, emax = pl.pallas_call(
        _minmax_body,
        grid=grid,
        in_specs=[pl.BlockSpec((bn, neig), lambda i: (i, 0))],
        out_specs=[_full((1, neig)), _full((1, neig))],
        out_shape=[jax.ShapeDtypeStruct((1, neig), jnp.float32)] * 2,
    )(eigenvectors)

    lam, lams, pb = pl.pallas_call(
        _lam_body,
        out_shape=[jax.ShapeDtypeStruct((neig, pd), jnp.float32)] * 2
        + [jax.ShapeDtypeStruct((1, pd), jnp.float32)],
    )(eigenvalues.reshape(neig, 1), alpha, emin.T, emax.T)

    h0 = pl.pallas_call(
        _h0_body,
        grid=grid,
        in_specs=[
            pl.BlockSpec((bn, in_dim), lambda i: (i, 0)),
            pl.BlockSpec((bn, neig), lambda i: (i, 0)),
            _full((hd, in_dim)),
            _full((1, hd)),
            _full((neig, pd)),
            _full((1, pd)),
        ],
        out_specs=pl.BlockSpec((bn, d), lambda i: (i, 0)),
        out_shape=jax.ShapeDtypeStruct((n, d), jnp.float32),
    )(x, eigenvectors, W_feat, b_feat.reshape(1, hd), lams, pb)

    e_pad = _NS * _SUP * math.ceil(e / (_NS * _SUP))
    src = jnp.concatenate([edge_index[0], jnp.zeros((e_pad - e,), jnp.int32)])
    dst = jnp.concatenate([edge_index[1], jnp.full((e_pad - e,), n, jnp.int32)])
    acc_rows = n // (_NC * _NB) + _L
    cs = -(-(acc_rows // _NS) // 8) * 8
    z2d = jnp.zeros((_CH, d), jnp.float32)
    z1d = jnp.zeros((cs,), jnp.float32)

    seg1 = _make_seg_sum(n, e_pad, d, with_count=True)
    seg2 = _make_seg_sum(n, e_pad, d, with_count=False)
    sums1, cnt = seg1(h0, src, dst, z2d, z1d)
    cnt2 = cnt.reshape(n, 1)

    def _one(r):
        return r[0] if isinstance(r, (list, tuple)) else r

    sage_specs = [
        pl.BlockSpec((bn, d), lambda i: (i, 0)),
        pl.BlockSpec((bn, 1), lambda i: (i, 0)),
        pl.BlockSpec((bn, d), lambda i: (i, 0)),
        _full((d, d)),
        _full((1, d)),
        _full((d, d)),
    ]
    h1 = pl.pallas_call(
        _sage_body,
        grid=grid,
        in_specs=sage_specs,
        out_specs=pl.BlockSpec((bn, d), lambda i: (i, 0)),
        out_shape=jax.ShapeDtypeStruct((n, d), jnp.float32),
    )(sums1, cnt2, h0, Wl1, bl1.reshape(1, d), Wr1)

    sums2 = _one(seg2(h1, src, dst, z2d))

    out = pl.pallas_call(
        _final_body,
        grid=grid,
        in_specs=sage_specs + [_full((out_dim, d)), _full((1, out_dim))],
        out_specs=pl.BlockSpec((bn, out_dim), lambda i: (i, 0)),
        out_shape=jax.ShapeDtypeStruct((n, out_dim), jnp.float32),
    )(sums2, cnt2, h1, Wl2, bl2.reshape(1, d), Wr2, W_out, b_out.reshape(1, out_dim))

    return (out, lam)


# speedup vs baseline: 2.3915x; 2.3915x over previous
import math

import jax
import jax.numpy as jnp
from jax import lax
from jax.experimental import pallas as pl
from jax.experimental.pallas import tpu as pltpu
from jax.experimental.pallas import tpu_sc as plsc

_NC = 2
_NS = 16
_L = 16
_CH = 256
_SUP = 2048
_R = 4
_NB = 1


def _chunks(total):
    out, off = [], 0
    while off < total:
        sz = min(_CH, total - off)
        out.append((off, sz))
        off += sz
    return out


def _make_seg_sum(n, e_pad, d, with_count, dt=jnp.bfloat16):
    nh = n // _NC
    nb = nh // _NB
    acc_rows = nb + _L
    per_tile = e_pad // _NS
    nsup = per_tile // _SUP
    ngrp = _SUP // _CH // _R
    cs = -(-(acc_rows // _NS) // 8) * 8
    cs_last = acc_rows - (_NS - 1) * cs
    cw_last = nb - (_NS - 1) * cs

    mesh = plsc.VectorSubcoreMesh(core_axis_name="c", subcore_axis_name="s")
    out_type = [jax.ShapeDtypeStruct((n, d), dt)]
    if with_count:
        out_type.append(jax.ShapeDtypeStruct((n,), jnp.float32))

    scratch = (
        [pltpu.VMEM((_SUP,), jnp.int32)] * 2
        + [pltpu.VMEM((_CH,), jnp.int32) for _ in range(_R)]
        + [pltpu.VMEM((_CH, d), dt) for _ in range(_R)]
        + [pltpu.SemaphoreType.DMA] * _R
        + [pltpu.SemaphoreType.DMA] * _R
        + [pltpu.VMEM_SHARED((acc_rows, d), dt)]
    )
    if with_count:
        scratch = scratch + (
            [pltpu.VMEM((_CH,), jnp.float32)]
            + [pltpu.SemaphoreType.DMA] * _R
            + [pltpu.VMEM((cs,), jnp.float32)]
            + [pltpu.VMEM_SHARED((acc_rows,), jnp.float32)]
        )

    def body(*refs):
        it = iter(refs)
        h_hbm = next(it)
        src_hbm = next(it)
        dst_hbm = next(it)
        z2d = next(it)
        z1d = next(it) if with_count else None
        sums_hbm = next(it)
        cnt_hbm = next(it) if with_count else None
        src_v = next(it)
        dst_v = next(it)
        locs = [next(it) for _ in range(_R)]
        rowss = [next(it) for _ in range(_R)]
        gsems = [next(it) for _ in range(_R)]
        ssems = [next(it) for _ in range(_R)]
        acc = next(it)
        if with_count:
            ones_v = next(it)
            csems = [next(it) for _ in range(_R)]
            wb1 = next(it)
            cntacc = next(it)

        c = lax.axis_index("c")
        s = lax.axis_index("s")

        if with_count:
            for i in range(_CH // _L):
                ones_v[pl.ds(i * _L, _L)] = jnp.ones((_L,), jnp.float32)

        def pass_body(p, carry):
            n0 = c * nh + p * nb

            pltpu.sync_copy(z2d, rowss[0])
            if with_count:
                pltpu.sync_copy(z1d, wb1)

            @pl.when(s < _NS - 1)
            def _():
                for off, sz in _chunks(cs):
                    pltpu.sync_copy(rowss[0].at[pl.ds(0, sz)],
                                    acc.at[pl.ds(s * cs + off, sz)])
                if with_count:
                    pltpu.sync_copy(wb1, cntacc.at[pl.ds(s * cs, cs)])

            @pl.when(s == _NS - 1)
            def _():
                for off, sz in _chunks(cs_last):
                    pltpu.sync_copy(rowss[0].at[pl.ds(0, sz)],
                                    acc.at[pl.ds((_NS - 1) * cs + off, sz)])
                if with_count:
                    pltpu.sync_copy(wb1.at[pl.ds(0, cs_last)],
                                    cntacc.at[pl.ds((_NS - 1) * cs, cs_last)])

            plsc.subcore_barrier()

            def sup_body(u, carry2):
                off = s * per_tile + u * _SUP
                pltpu.sync_copy(src_hbm.at[pl.ds(off, _SUP)], src_v)
                pltpu.sync_copy(dst_hbm.at[pl.ds(off, _SUP)], dst_v)
                for g in range(ngrp):
                    gds = []
                    for b in range(_R):
                        j = g * _R + b
                        def _drain():
                            pltpu.make_async_copy(
                                rowss[b], acc.at[locs[b]], ssems[b]).wait()
                            if with_count:
                                pltpu.make_async_copy(
                                    ones_v, cntacc.at[locs[b]], csems[b]).wait()
                        if g == 0:
                            pl.when(u > 0)(_drain)
                        else:
                            _drain()
                        for i in range(_CH // _L):
                            o = j * _CH + i * _L
                            dvec = dst_v[pl.ds(o, _L)]
                            m = (dvec >= n0) & (dvec < n0 + nb)
                            locs[b][pl.ds(i * _L, _L)] = jnp.where(m, dvec - n0, nb)
                        gds.append(pltpu.async_copy(
                            h_hbm.at[src_v.at[pl.ds(j * _CH, _CH)]],
                            rowss[b], gsems[b]))
                    for b in range(_R):
                        gds[b].wait()
                        pltpu.async_copy(rowss[b], acc.at[locs[b]], ssems[b],
                                         add=True)
                        if with_count:
                            pltpu.async_copy(ones_v, cntacc.at[locs[b]],
                                             csems[b], add=True)
                return carry2

            lax.fori_loop(0, nsup, sup_body, 0)

            for b in range(_R):
                pltpu.make_async_copy(rowss[b], acc.at[locs[b]], ssems[b]).wait()
                if with_count:
                    pltpu.make_async_copy(ones_v, cntacc.at[locs[b]],
                                          csems[b]).wait()
            plsc.subcore_barrier()

            @pl.when(s < _NS - 1)
            def _():
                for off, sz in _chunks(cs):
                    pltpu.sync_copy(acc.at[pl.ds(s * cs + off, sz)],
                                    rowss[0].at[pl.ds(0, sz)])
                    pltpu.sync_copy(rowss[0].at[pl.ds(0, sz)],
                                    sums_hbm.at[pl.ds(n0 + s * cs + off, sz)])
                if with_count:
                    pltpu.sync_copy(cntacc.at[pl.ds(s * cs, cs)], wb1)
                    pltpu.sync_copy(wb1, cnt_hbm.at[pl.ds(n0 + s * cs, cs)])

            @pl.when(s == _NS - 1)
            def _():
                for off, sz in _chunks(cw_last):
                    pltpu.sync_copy(acc.at[pl.ds((_NS - 1) * cs + off, sz)],
                                    rowss[0].at[pl.ds(0, sz)])
                    pltpu.sync_copy(
                        rowss[0].at[pl.ds(0, sz)],
                        sums_hbm.at[pl.ds(n0 + (_NS - 1) * cs + off, sz)])
                if with_count:
                    pltpu.sync_copy(cntacc.at[pl.ds((_NS - 1) * cs, cw_last)],
                                    wb1.at[pl.ds(0, cw_last)])
                    pltpu.sync_copy(
                        wb1.at[pl.ds(0, cw_last)],
                        cnt_hbm.at[pl.ds(n0 + (_NS - 1) * cs, cw_last)])
            return carry

        lax.fori_loop(0, _NB, pass_body, 0)

    return pl.kernel(
        body, out_type=out_type, mesh=mesh, scratch_types=scratch,
        compiler_params=pltpu.CompilerParams(use_tc_tiling_on_sc=False),
    )


def _minmax_body(ev_ref, min_ref, max_ref):
    i = pl.program_id(0)
    bmin = jnp.min(ev_ref[...], axis=0, keepdims=True)
    bmax = jnp.max(ev_ref[...], axis=0, keepdims=True)

    @pl.when(i == 0)
    def _():
        min_ref[...] = bmin
        max_ref[...] = bmax

    @pl.when(i > 0)
    def _():
        min_ref[...] = jnp.minimum(min_ref[...], bmin)
        max_ref[...] = jnp.maximum(max_ref[...], bmax)


def _lam_body(evals_ref, alpha_ref, emin_ref, emax_ref,
              lam_ref, lams_ref, pb_ref):
    k = alpha_ref.shape[0]
    ev = evals_ref[...]
    lmin = jnp.min(ev)
    lmax = jnp.max(ev)
    x = (ev - lmin) / (lmax - lmin) * 2.0 - 1.0
    acc = jnp.ones_like(x) * alpha_ref[0:1, :] + x * alpha_ref[1:2, :]

    def step(kk, carry):
        tkm1, tk, a = carry
        tk1 = 2.0 * x * tk - tkm1
        a = a + tk1 * alpha_ref[pl.ds(kk, 1), :]
        return (tk, tk1, a)

    _, _, acc = lax.fori_loop(2, k, step, (jnp.ones_like(x), x, acc))
    lam_ref[...] = acc
    emin = emin_ref[...]
    emax = emax_ref[...]
    a_col = 2.0 / (emax - emin)
    b_col = -2.0 * emin / (emax - emin) - 1.0
    lams_ref[...] = acc * a_col
    pb_ref[...] = jnp.sum(acc * b_col, axis=0, keepdims=True)


def _h0_body(x_ref, ev_ref, wf_ref, bf_ref, lams_ref, pb_ref, h0_ref):
    feat = lax.dot_general(x_ref[...], wf_ref[...], (((1,), (1,)), ((), ())),
                           preferred_element_type=jnp.float32) + bf_ref[...]
    pos = lax.dot_general(ev_ref[...], lams_ref[...], (((1,), (0,)), ((), ())),
                          preferred_element_type=jnp.float32) + pb_ref[...]
    h0_ref[...] = jnp.concatenate([feat, pos], axis=1)


def _sage_body(sums_ref, cnt_ref, h_ref, wl_ref, bl_ref, wr_ref, out_ref):
    agg = sums_ref[...].astype(jnp.float32) * (
        1.0 / jnp.maximum(cnt_ref[...], 1.0))
    t = (lax.dot_general(agg, wl_ref[...], (((1,), (1,)), ((), ())),
                         preferred_element_type=jnp.float32) + bl_ref[...]
         + lax.dot_general(h_ref[...], wr_ref[...], (((1,), (1,)), ((), ())),
                           preferred_element_type=jnp.float32))
    out_ref[...] = jnp.maximum(t, 0.0)


def _final_body(sums_ref, cnt_ref, h_ref, wl_ref, bl_ref, wr_ref,
                wo_ref, bo_ref, out_ref):
    agg = sums_ref[...].astype(jnp.float32) * (
        1.0 / jnp.maximum(cnt_ref[...], 1.0))
    t = (lax.dot_general(agg, wl_ref[...], (((1,), (1,)), ((), ())),
                         preferred_element_type=jnp.float32) + bl_ref[...]
         + lax.dot_general(h_ref[...], wr_ref[...], (((1,), (1,)), ((), ())),
                           preferred_element_type=jnp.float32))
    h2 = jnp.maximum(t, 0.0)
    z = lax.dot_general(h2, wo_ref[...], (((1,), (1,)), ((), ())),
                        preferred_element_type=jnp.float32) + bo_ref[...]
    zs = z - jnp.max(z, axis=1, keepdims=True)
    out_ref[...] = zs - jnp.log(jnp.sum(jnp.exp(zs), axis=1, keepdims=True))


def _full(shape):
    return pl.BlockSpec(shape, lambda i: (0,) * len(shape))


def kernel(x, eigenvectors, eigenvalues, edge_index, W_feat, b_feat, alpha,
           Wl1, bl1, Wr1, Wl2, bl2, Wr2, W_out, b_out):
    n, in_dim = x.shape
    neig = eigenvectors.shape[1]
    hd = W_feat.shape[0]
    pd = alpha.shape[1]
    d = hd + pd
    out_dim = W_out.shape[0]
    e = edge_index.shape[1]

    bn = 2000
    grid = (n // bn,)

    emin, emax = pl.pallas_call(
        _minmax_body,
        grid=grid,
        in_specs=[pl.BlockSpec((bn, neig), lambda i: (i, 0))],
        out_specs=[_full((1, neig)), _full((1, neig))],
        out_shape=[jax.ShapeDtypeStruct((1, neig), jnp.float32)] * 2,
    )(eigenvectors)

    lam, lams, pb = pl.pallas_call(
        _lam_body,
        out_shape=[jax.ShapeDtypeStruct((neig, pd), jnp.float32)] * 2
        + [jax.ShapeDtypeStruct((1, pd), jnp.float32)],
    )(eigenvalues.reshape(neig, 1), alpha, emin.T, emax.T)

    h0 = pl.pallas_call(
        _h0_body,
        grid=grid,
        in_specs=[
            pl.BlockSpec((bn, in_dim), lambda i: (i, 0)),
            pl.BlockSpec((bn, neig), lambda i: (i, 0)),
            _full((hd, in_dim)),
            _full((1, hd)),
            _full((neig, pd)),
            _full((1, pd)),
        ],
        out_specs=pl.BlockSpec((bn, d), lambda i: (i, 0)),
        out_shape=jax.ShapeDtypeStruct((n, d), jnp.float32),
    )(x, eigenvectors, W_feat, b_feat.reshape(1, hd), lams, pb)

    e_pad = _NS * _SUP * math.ceil(e / (_NS * _SUP))
    src = jnp.concatenate([edge_index[0], jnp.zeros((e_pad - e,), jnp.int32)])
    dst = jnp.concatenate([edge_index[1], jnp.full((e_pad - e,), n, jnp.int32)])
    acc_rows = n // (_NC * _NB) + _L
    cs = -(-(acc_rows // _NS) // 8) * 8
    z2d = jnp.zeros((_CH, d), jnp.bfloat16)
    z1d = jnp.zeros((cs,), jnp.float32)

    seg1 = _make_seg_sum(n, e_pad, d, with_count=True)
    seg2 = _make_seg_sum(n, e_pad, d, with_count=False)
    sums1, cnt = seg1(h0.astype(jnp.bfloat16), src, dst, z2d, z1d)
    cnt2 = cnt.reshape(n, 1)

    def _one(r):
        return r[0] if isinstance(r, (list, tuple)) else r

    sage_specs = [
        pl.BlockSpec((bn, d), lambda i: (i, 0)),
        pl.BlockSpec((bn, 1), lambda i: (i, 0)),
        pl.BlockSpec((bn, d), lambda i: (i, 0)),
        _full((d, d)),
        _full((1, d)),
        _full((d, d)),
    ]
    h1 = pl.pallas_call(
        _sage_body,
        grid=grid,
        in_specs=sage_specs,
        out_specs=pl.BlockSpec((bn, d), lambda i: (i, 0)),
        out_shape=jax.ShapeDtypeStruct((n, d), jnp.float32),
    )(sums1, cnt2, h0, Wl1, bl1.reshape(1, d), Wr1)

    sums2 = _one(seg2(h1.astype(jnp.bfloat16), src, dst, z2d))

    out = pl.pallas_call(
        _final_body,
        grid=grid,
        in_specs=sage_specs + [_full((out_dim, d)), _full((1, out_dim))],
        out_specs=pl.BlockSpec((bn, out_dim), lambda i: (i, 0)),
        out_shape=jax.ShapeDtypeStruct((n, out_dim), jnp.float32),
    )(sums2, cnt2, h1, Wl2, bl2.reshape(1, d), Wr2, W_out, b_out.reshape(1, out_dim))

    return (out, lam)
